# baseline (speedup 1.0000x reference)
import functools

import jax
import jax.numpy as jnp
from jax.experimental import pallas as pl
from jax.experimental.pallas import tpu as pltpu

B, S, D = 8, 512, 768
SPAN_NUM = 2048
MAX_W = 4
D_FF = 3072
N_CLS = 3

M_TILE = 1024
NT = SPAN_NUM // M_TILE
LANES = 128
LN_EPS = 1e-12


def _gelu_exact(x):
    return 0.5 * x * (1.0 + jax.lax.erf(x * 0.7071067811865476))


def _fused_body(p_ref, x_ref, wi_f, bi_f, wo_f, bo_f,
                wi_r, bi_r, wo_r, bo_r, wg, u, cb, out_ref):
    p = p_ref[0, 0]
    start = p[0:1, :]
    end = p[1:2, :]
    invw = p[2:3, :]
    xb = x_ref[0]

    s_iota = jax.lax.broadcasted_iota(jnp.int32, (S, M_TILE), 0).astype(jnp.float32)
    sel = jnp.logical_and(s_iota >= start, s_iota < end)
    at = jnp.where(sel, invw, 0.0).astype(jnp.bfloat16)

    e = jax.lax.dot_general(at, xb, (((0,), (0,)), ((), ())),
                            preferred_element_type=jnp.float32)
    e_bf = e.astype(jnp.bfloat16)

    def decoder(wi, bi, wo, bo):
        t = jnp.dot(e_bf, wi[...], preferred_element_type=jnp.float32) + bi[...]
        h = (t + t * jax.lax.erf(t)).astype(jnp.bfloat16)
        o = jnp.dot(h, wo[...], preferred_element_type=jnp.float32) + bo[...]
        y = o + e
        m = jnp.mean(y, axis=-1, keepdims=True)
        s2 = jnp.mean(y * y, axis=-1, keepdims=True)
        inv = jax.lax.rsqrt(jnp.maximum(s2 - m * m, 0.0) + LN_EPS)
        return y.astype(jnp.bfloat16), m, inv

    y_f, m_f, i_f = decoder(wi_f, bi_f, wo_f, bo_f)
    y_r, m_r, i_r = decoder(wi_r, bi_r, wo_r, bo_r)

    z_f = (jnp.dot(y_f, wg[0], preferred_element_type=jnp.float32)
           - m_f * u[0]) * i_f
    z_r = (jnp.dot(y_r, wg[1], preferred_element_type=jnp.float32)
           - m_r * u[1]) * i_r
    out_ref[...] = z_f + z_r + cb[...]


@jax.jit
def _fused(p, x, wi_f, bi_f, wo_f, bo_f,
           wi_r, bi_r, wo_r, bo_r, wg, u, cb):
    full = lambda shape: pl.BlockSpec(shape, lambda b, t: (0,) * len(shape))
    grid = (B, NT)
    return pl.pallas_call(
        _fused_body,
        grid=grid,
        in_specs=[
            pl.BlockSpec((1, 1, 8, M_TILE), lambda b, t: (b, t, 0, 0)),
            pl.BlockSpec((1, S, D), lambda b, t: (b, 0, 0)),
            full((D, D_FF)), full((1, D_FF)), full((D_FF, D)), full((1, D)),
            full((D, D_FF)), full((1, D_FF)), full((D_FF, D)), full((1, D)),
            full((2, D, LANES)), full((2, 1, LANES)), full((1, LANES)),
        ],
        out_specs=pl.BlockSpec((M_TILE, LANES), lambda b, t: (b * NT + t, 0)),
        out_shape=jax.ShapeDtypeStruct((B * SPAN_NUM, LANES), jnp.float32),
        compiler_params=pltpu.CompilerParams(
            dimension_semantics=("parallel", "parallel")),
    )(p, x, wi_f, bi_f, wo_f, bo_f, wi_r, bi_r, wo_r, bo_r, wg, u, cb)


def kernel(input_bert_features, attention_mask, spans, span_mask,
           related_spans_tensor, sentence_length, Wi_f, bi_f, Wo_f, bo_f,
           g_f, be_f, Wi_r, bi_r, Wo_r, bo_r, g_r, be_r, Wa, ba, Wop, bop):
    start = spans[..., 0].astype(jnp.float32)
    width = spans[..., 2].astype(jnp.float32)
    end = start + width
    invw = span_mask.astype(jnp.float32) / jnp.maximum(width, 1.0)
    pack = jnp.stack([start, end, invw], axis=-2)
    p = jnp.zeros((B, 8, SPAN_NUM), jnp.float32).at[:, :3, :].set(pack)
    p = p.reshape(B, 8, NT, M_TILE).transpose(0, 2, 1, 3)

    wc = jnp.zeros((2, D, LANES), jnp.float32)
    wc = wc.at[0, :, :N_CLS].set(Wa).at[1, :, N_CLS:2 * N_CLS].set(Wop)
    wg = wc * jnp.stack([g_f, g_r])[:, :, None]
    u = jnp.sum(wg, axis=1, keepdims=True)
    cb = (be_f @ wc[0] + be_r @ wc[1]).reshape(1, LANES)
    cb = cb.at[0, :N_CLS].add(ba).at[0, N_CLS:2 * N_CLS].add(bop)

    bf = jnp.bfloat16
    c = 0.7071067811865476
    out = _fused(p, input_bert_features.astype(bf),
                 (Wi_f * c).astype(bf), (bi_f * c).reshape(1, D_FF),
                 (Wo_f * c).astype(bf), bo_f.reshape(1, D),
                 (Wi_r * c).astype(bf), (bi_r * c).reshape(1, D_FF),
                 (Wo_r * c).astype(bf), bo_r.reshape(1, D),
                 wg.astype(bf), u, cb)
    return out[:, :2 * N_CLS].reshape(B, SPAN_NUM, 2 * N_CLS)


# speedup vs baseline: 1.0317x; 1.0317x over previous
import functools

import jax
import jax.numpy as jnp
from jax.experimental import pallas as pl
from jax.experimental.pallas import tpu as pltpu

B, S, D = 8, 512, 768
SPAN_NUM = 2048
MAX_W = 4
D_FF = 3072
N_CLS = 3

M_TILE = 1024
NT = SPAN_NUM // M_TILE
LANES = 128
LN_EPS = 1e-12


def _gelu_exact(x):
    return 0.5 * x * (1.0 + jax.lax.erf(x * 0.7071067811865476))


def _fused_body(p_ref, x_ref, wi_f, bi_f, wo_f, bo_f,
                wi_r, bi_r, wo_r, bo_r, wg, u, cb, out_ref):
    p = p_ref[0, 0]
    start = p[0:1, :]
    end = p[1:2, :]
    invw = p[2:3, :]
    xb = x_ref[0]

    s_iota = jax.lax.broadcasted_iota(jnp.int32, (S, M_TILE), 0).astype(jnp.float32)
    sel = jnp.logical_and(s_iota >= start, s_iota < end)
    at = jnp.where(sel, invw, 0.0).astype(jnp.bfloat16)

    e = jax.lax.dot_general(at, xb, (((0,), (0,)), ((), ())),
                            preferred_element_type=jnp.float32)
    e_bf = e.astype(jnp.bfloat16)

    def decoder(wi, bi, wo, bo):
        nc = D_FF // D
        acc = None
        for k in range(nc):
            t = (jnp.dot(e_bf, wi[:, k * D:(k + 1) * D],
                         preferred_element_type=jnp.float32)
                 + bi[:, k * D:(k + 1) * D])
            h = (t + t * jax.lax.erf(t)).astype(jnp.bfloat16)
            part = jnp.dot(h, wo[k * D:(k + 1) * D, :],
                           preferred_element_type=jnp.float32)
            acc = part if acc is None else acc + part
        y = acc + bo[...] + e
        m = jnp.mean(y, axis=-1, keepdims=True)
        s2 = jnp.mean(y * y, axis=-1, keepdims=True)
        inv = jax.lax.rsqrt(jnp.maximum(s2 - m * m, 0.0) + LN_EPS)
        return y.astype(jnp.bfloat16), m, inv

    y_f, m_f, i_f = decoder(wi_f, bi_f, wo_f, bo_f)
    y_r, m_r, i_r = decoder(wi_r, bi_r, wo_r, bo_r)

    z_f = (jnp.dot(y_f, wg[0], preferred_element_type=jnp.float32)
           - m_f * u[0]) * i_f
    z_r = (jnp.dot(y_r, wg[1], preferred_element_type=jnp.float32)
           - m_r * u[1]) * i_r
    out_ref[...] = z_f + z_r + cb[...]


@jax.jit
def _fused(p, x, wi_f, bi_f, wo_f, bo_f,
           wi_r, bi_r, wo_r, bo_r, wg, u, cb):
    full = lambda shape: pl.BlockSpec(shape, lambda b, t: (0,) * len(shape))
    grid = (B, NT)
    return pl.pallas_call(
        _fused_body,
        grid=grid,
        in_specs=[
            pl.BlockSpec((1, 1, 8, M_TILE), lambda b, t: (b, t, 0, 0)),
            pl.BlockSpec((1, S, D), lambda b, t: (b, 0, 0)),
            full((D, D_FF)), full((1, D_FF)), full((D_FF, D)), full((1, D)),
            full((D, D_FF)), full((1, D_FF)), full((D_FF, D)), full((1, D)),
            full((2, D, LANES)), full((2, 1, LANES)), full((1, LANES)),
        ],
        out_specs=pl.BlockSpec((M_TILE, LANES), lambda b, t: (b * NT + t, 0)),
        out_shape=jax.ShapeDtypeStruct((B * SPAN_NUM, LANES), jnp.float32),
        compiler_params=pltpu.CompilerParams(
            dimension_semantics=("parallel", "parallel")),
    )(p, x, wi_f, bi_f, wo_f, bo_f, wi_r, bi_r, wo_r, bo_r, wg, u, cb)


def kernel(input_bert_features, attention_mask, spans, span_mask,
           related_spans_tensor, sentence_length, Wi_f, bi_f, Wo_f, bo_f,
           g_f, be_f, Wi_r, bi_r, Wo_r, bo_r, g_r, be_r, Wa, ba, Wop, bop):
    start = spans[..., 0].astype(jnp.float32)
    width = spans[..., 2].astype(jnp.float32)
    end = start + width
    invw = span_mask.astype(jnp.float32) / jnp.maximum(width, 1.0)
    pack = jnp.stack([start, end, invw], axis=-2)
    p = jnp.zeros((B, 8, SPAN_NUM), jnp.float32).at[:, :3, :].set(pack)
    p = p.reshape(B, 8, NT, M_TILE).transpose(0, 2, 1, 3)

    wc = jnp.zeros((2, D, LANES), jnp.float32)
    wc = wc.at[0, :, :N_CLS].set(Wa).at[1, :, N_CLS:2 * N_CLS].set(Wop)
    wg = wc * jnp.stack([g_f, g_r])[:, :, None]
    u = jnp.sum(wg, axis=1, keepdims=True)
    cb = (be_f @ wc[0] + be_r @ wc[1]).reshape(1, LANES)
    cb = cb.at[0, :N_CLS].add(ba).at[0, N_CLS:2 * N_CLS].add(bop)

    bf = jnp.bfloat16
    c = 0.7071067811865476
    out = _fused(p, input_bert_features.astype(bf),
                 (Wi_f * c).astype(bf), (bi_f * c).reshape(1, D_FF),
                 (Wo_f * c).astype(bf), bo_f.reshape(1, D),
                 (Wi_r * c).astype(bf), (bi_r * c).reshape(1, D_FF),
                 (Wo_r * c).astype(bf), bo_r.reshape(1, D),
                 wg.astype(bf), u, cb)
    return out[:, :2 * N_CLS].reshape(B, SPAN_NUM, 2 * N_CLS)
